# baseline (speedup 1.0000x reference)
import functools

import jax
import jax.numpy as jnp
from jax import lax
from jax.experimental import pallas as pl
from jax.experimental.pallas import tpu as pltpu
from jax.experimental.pallas import tpu_sc as plsc

_N = 10000
_E = 320000
_D = 128
_H = 128
_O = 64

_NC = 2
_NT = 16
_NW = _NC * _NT

_CHUNK = 128
_N_PAD = 10240
_E_PAD = 327680
_CH_W = _E_PAD // (_NW * _CHUNK)
_SCH = 8
_NSCH = _CH_W // _SCH
_ZR = _N_PAD // _NT

_RB = 2000
_NSUB = 4


def _make_sc_scatter(dcol, s0, s1, nbuf=2):
  mesh = plsc.VectorSubcoreMesh(core_axis_name="c", subcore_axis_name="s")

  @functools.partial(
      pl.kernel,
      out_type=jax.ShapeDtypeStruct((_NC, _N_PAD, dcol), jnp.float32),
      mesh=mesh,
      compiler_params=pltpu.CompilerParams(use_tc_tiling_on_sc=False),
      scratch_types=[
          pltpu.VMEM((_SCH, _CHUNK), jnp.int32),
          pltpu.VMEM((_SCH, _CHUNK), jnp.int32),
          [pltpu.VMEM((_CHUNK, dcol), jnp.float32) for _ in range(nbuf)],
          pltpu.VMEM_SHARED((_N_PAD, dcol), jnp.float32),
          [pltpu.SemaphoreType.DMA for _ in range(nbuf)],
      ],
  )
  def scat(y_hbm, src_hbm, dst_hbm, z_hbm, out_hbm,
           src_v, dst_v, bufs, acc, sems):
    c = lax.axis_index("c")
    s = lax.axis_index("s")
    nsch = jnp.where(c == 0, s0, s1)
    row0 = jnp.where(c == 0, s * s0, 16 * s0 + s * s1) * _SCH
    with jax.named_scope("acc_zero"):
      pltpu.sync_copy(z_hbm, bufs[0])
      for k in range(_ZR // _CHUNK):
        pltpu.sync_copy(bufs[0], acc.at[pl.ds(s * _ZR + k * _CHUNK, _CHUNK)])
      plsc.subcore_barrier()

    def fire(j, buf, sem):
      for q in range(_NSUB):
        r = q * (_CHUNK // _NSUB)
        pltpu.async_copy(y_hbm.at[src_v.at[j, pl.ds(r, _CHUNK // _NSUB)]],
                         buf.at[pl.ds(r, _CHUNK // _NSUB)], sem)

    def outer(g, carry):
      base = row0 + g * _SCH
      pltpu.sync_copy(src_hbm.at[pl.ds(base, _SCH)], src_v)
      pltpu.sync_copy(dst_hbm.at[pl.ds(base, _SCH)], dst_v)
      for b in range(nbuf):
        fire(b, bufs[b], sems[b])

      def step(i, c2):
        j0 = i * nbuf
        for b in range(nbuf):
          j = j0 + b
          pltpu.make_async_copy(y_hbm.at[src_v.at[j]], bufs[b], sems[b]).wait()
          pltpu.sync_copy(bufs[b], acc.at[dst_v.at[j]], add=True)

          @pl.when(j + nbuf < _SCH)
          def _(b=b, j=j):
            fire(j + nbuf, bufs[b], sems[b])
        return c2

      lax.fori_loop(0, _SCH // nbuf, step, 0)
      return carry

    with jax.named_scope("edge_loop"):
      lax.fori_loop(0, nsch, outer, 0)
      plsc.subcore_barrier()
    with jax.named_scope("acc_writeout"):
      for k in range(_ZR // _CHUNK):
        r = s * _ZR + k * _CHUNK
        pltpu.sync_copy(acc.at[pl.ds(r, _CHUNK)], out_hbm.at[c, pl.ds(r, _CHUNK)])

  return scat


_sc_scatter_h = _make_sc_scatter(_H, 17, 3, nbuf=2)
_sc_scatter_o = _make_sc_scatter(_O, 18, 2, nbuf=4)


def _make_sc_degree():
  mesh = plsc.VectorSubcoreMesh(core_axis_name="c", subcore_axis_name="s")

  @functools.partial(
      pl.kernel,
      out_type=jax.ShapeDtypeStruct((_NC, _N_PAD, 16), jnp.float32),
      mesh=mesh,
      compiler_params=pltpu.CompilerParams(use_tc_tiling_on_sc=False),
      scratch_types=[
          pltpu.VMEM((_CH_W, _CHUNK), jnp.int32),
          pltpu.VMEM((_CHUNK, 16), jnp.float32),
          pltpu.VMEM((_CHUNK, 16), jnp.float32),
          pltpu.VMEM_SHARED((_N_PAD, 16), jnp.float32),
      ],
  )
  def degk(dst_hbm, ones_hbm, z_hbm, out_hbm, dst_v, ones_v, z_v, acc):
    c = lax.axis_index("c")
    s = lax.axis_index("s")
    row0 = (c * _NT + s) * _CH_W
    pltpu.sync_copy(dst_hbm.at[pl.ds(row0, _CH_W)], dst_v)
    pltpu.sync_copy(ones_hbm, ones_v)
    pltpu.sync_copy(z_hbm, z_v)
    for k in range(_ZR // _CHUNK):
      pltpu.sync_copy(z_v, acc.at[pl.ds(s * _ZR + k * _CHUNK, _CHUNK)])
    plsc.subcore_barrier()

    def step(j, carry):
      pltpu.sync_copy(ones_v, acc.at[dst_v.at[j]], add=True)
      return carry

    lax.fori_loop(0, _CH_W, step, 0)
    plsc.subcore_barrier()
    for k in range(_ZR // _CHUNK):
      r = s * _ZR + k * _CHUNK
      pltpu.sync_copy(acc.at[pl.ds(r, _CHUNK)], out_hbm.at[c, pl.ds(r, _CHUNK)])

  return degk


_sc_degree = _make_sc_degree()


def _mm_body(x_ref, w_ref, o_ref):
  o_ref[...] = jnp.dot(x_ref[...], w_ref[...],
                       preferred_element_type=jnp.float32)


def _scale1_body(dp0_ref, dp1_ref, xw_ref, dinv_ref, y_ref):
  deg = dp0_ref[...][:, 0:1] + dp1_ref[...][:, 0:1] + 1.0
  dinv = lax.rsqrt(deg)
  dinv_ref[...] = dinv
  y_ref[...] = xw_ref[...] * dinv


def _layer2_body(a0_ref, a1_ref, xw_ref, dinv_ref, b1_ref, w2_ref, y2_ref):
  dinv = dinv_ref[...]
  h = dinv * (a0_ref[...] + a1_ref[...]) + (dinv * dinv) * xw_ref[...]
  h = jnp.maximum(h + b1_ref[...], 0.0)
  z = jnp.dot(h, w2_ref[...], preferred_element_type=jnp.float32)
  y2_ref[...] = dinv * z


def _final_body(a0_ref, a1_ref, y2_ref, dinv_ref, b2_ref, o_ref):
  o = dinv_ref[...] * (a0_ref[...] + a1_ref[...] + y2_ref[...]) + b2_ref[...]
  m = jnp.max(o, axis=1, keepdims=True)
  lse = jnp.log(jnp.sum(jnp.exp(o - m), axis=1, keepdims=True)) + m
  o_ref[...] = o - lse


def _rows(shape):
  return pl.BlockSpec(shape, lambda i: (i, 0))


def kernel(x, edge_index, W1, b1, W2, b2):
  src = edge_index[0].astype(jnp.int32)
  dst = edge_index[1].astype(jnp.int32)
  pad = _E_PAD - _E
  src2d = jnp.concatenate([src, jnp.zeros((pad,), jnp.int32)]).reshape(-1, _CHUNK)
  dst2d = jnp.concatenate([dst, jnp.full((pad,), _N, jnp.int32)]).reshape(-1, _CHUNK)

  z_h = jnp.zeros((_CHUNK, _H), jnp.float32)
  z_o = jnp.zeros((_CHUNK, _O), jnp.float32)
  z16 = jnp.zeros((_CHUNK, 16), jnp.float32)
  ones16 = jnp.ones((_CHUNK, 16), jnp.float32)

  grid = (_N // _RB,)

  degp = _sc_degree(dst2d, ones16, z16)

  xw1 = pl.pallas_call(
      _mm_body, grid=grid,
      in_specs=[_rows((_RB, _D)), pl.BlockSpec((_D, _H), lambda i: (0, 0))],
      out_specs=_rows((_RB, _H)),
      out_shape=jax.ShapeDtypeStruct((_N, _H), jnp.float32),
  )(x, W1)

  dinv, y1 = pl.pallas_call(
      _scale1_body, grid=grid,
      in_specs=[_rows((_RB, 16)), _rows((_RB, 16)), _rows((_RB, _H))],
      out_specs=(_rows((_RB, 1)), _rows((_RB, _H))),
      out_shape=(jax.ShapeDtypeStruct((_N, 1), jnp.float32),
                 jax.ShapeDtypeStruct((_N, _H), jnp.float32)),
  )(degp[0, :_N], degp[1, :_N], xw1)

  acc1 = _sc_scatter_h(y1, src2d, dst2d, z_h)

  y2 = pl.pallas_call(
      _layer2_body, grid=grid,
      in_specs=[_rows((_RB, _H)), _rows((_RB, _H)), _rows((_RB, _H)),
                _rows((_RB, 1)), pl.BlockSpec((1, _H), lambda i: (0, 0)),
                pl.BlockSpec((_H, _O), lambda i: (0, 0))],
      out_specs=_rows((_RB, _O)),
      out_shape=jax.ShapeDtypeStruct((_N, _O), jnp.float32),
  )(acc1[0, :_N], acc1[1, :_N], xw1, dinv, b1.reshape(1, _H), W2)

  acc2 = _sc_scatter_o(y2, src2d, dst2d, z_o)

  out = pl.pallas_call(
      _final_body, grid=grid,
      in_specs=[_rows((_RB, _O)), _rows((_RB, _O)), _rows((_RB, _O)),
                _rows((_RB, 1)), pl.BlockSpec((1, _O), lambda i: (0, 0))],
      out_specs=_rows((_RB, _O)),
      out_shape=jax.ShapeDtypeStruct((_N, _O), jnp.float32),
  )(acc2[0, :_N], acc2[1, :_N], y2, dinv, b2.reshape(1, _O))
  return out


# speedup vs baseline: 1.1104x; 1.1104x over previous
import functools

import jax
import jax.numpy as jnp
from jax import lax
from jax.experimental import pallas as pl
from jax.experimental.pallas import tpu as pltpu
from jax.experimental.pallas import tpu_sc as plsc

_N = 10000
_E = 320000
_D = 128
_H = 128
_O = 64

_NC = 2
_NT = 16
_NW = _NC * _NT

_CHUNK = 128
_N_PAD = 10240
_E_PAD = 327680
_CH_W = _E_PAD // (_NW * _CHUNK)
_SCH = 8
_NSCH = _CH_W // _SCH
_ZR = _N_PAD // _NT

_RB = 2000
_NSUB = 4


def _make_sc_scatter(dcol, s0, s1, nbuf=2):
  mesh = plsc.VectorSubcoreMesh(core_axis_name="c", subcore_axis_name="s")

  @functools.partial(
      pl.kernel,
      out_type=jax.ShapeDtypeStruct((_NC, _N_PAD, dcol), jnp.float32),
      mesh=mesh,
      compiler_params=pltpu.CompilerParams(use_tc_tiling_on_sc=False),
      scratch_types=[
          pltpu.VMEM((_SCH, _CHUNK), jnp.int32),
          pltpu.VMEM((_SCH, _CHUNK), jnp.int32),
          [pltpu.VMEM((_CHUNK, dcol), jnp.float32) for _ in range(nbuf)],
          pltpu.VMEM_SHARED((_N_PAD, dcol), jnp.float32),
          [pltpu.SemaphoreType.DMA for _ in range(nbuf)],
      ],
  )
  def scat(y_hbm, src_hbm, dst_hbm, z_hbm, out_hbm,
           src_v, dst_v, bufs, acc, sems):
    c = lax.axis_index("c")
    s = lax.axis_index("s")
    nsch = jnp.where(c == 0, s0, s1)
    row0 = jnp.where(c == 0, s * s0, 16 * s0 + s * s1) * _SCH
    with jax.named_scope("acc_zero"):
      pltpu.sync_copy(z_hbm, bufs[0])
      for k in range(_ZR // _CHUNK):
        pltpu.sync_copy(bufs[0], acc.at[pl.ds(s * _ZR + k * _CHUNK, _CHUNK)])
      plsc.subcore_barrier()

    def fire(j, buf, sem):
      for q in range(_NSUB):
        r = q * (_CHUNK // _NSUB)
        pltpu.async_copy(y_hbm.at[src_v.at[j, pl.ds(r, _CHUNK // _NSUB)]],
                         buf.at[pl.ds(r, _CHUNK // _NSUB)], sem)

    def outer(g, carry):
      base = row0 + g * _SCH
      pltpu.sync_copy(src_hbm.at[pl.ds(base, _SCH)], src_v)
      pltpu.sync_copy(dst_hbm.at[pl.ds(base, _SCH)], dst_v)
      for b in range(nbuf):
        fire(b, bufs[b], sems[b])

      def step(i, c2):
        j0 = i * nbuf
        for b in range(nbuf):
          j = j0 + b
          pltpu.make_async_copy(y_hbm.at[src_v.at[j]], bufs[b], sems[b]).wait()
          pltpu.sync_copy(bufs[b], acc.at[dst_v.at[j]], add=True)

          @pl.when(j + nbuf < _SCH)
          def _(b=b, j=j):
            fire(j + nbuf, bufs[b], sems[b])
        return c2

      lax.fori_loop(0, _SCH // nbuf, step, 0)
      return carry

    with jax.named_scope("edge_loop"):
      lax.fori_loop(0, nsch, outer, 0)
      plsc.subcore_barrier()
    with jax.named_scope("acc_writeout"):
      for k in range(_ZR // _CHUNK):
        r = s * _ZR + k * _CHUNK
        pltpu.sync_copy(acc.at[pl.ds(r, _CHUNK)], out_hbm.at[c, pl.ds(r, _CHUNK)])

  return scat


_sc_scatter_h = _make_sc_scatter(_H, 19, 1, nbuf=2)
_sc_scatter_o = _make_sc_scatter(_O, 18, 2, nbuf=4)


def _make_sc_degree():
  mesh = plsc.VectorSubcoreMesh(core_axis_name="c", subcore_axis_name="s")

  @functools.partial(
      pl.kernel,
      out_type=jax.ShapeDtypeStruct((_NC, _N_PAD, 16), jnp.float32),
      mesh=mesh,
      compiler_params=pltpu.CompilerParams(use_tc_tiling_on_sc=False),
      scratch_types=[
          pltpu.VMEM((_CH_W, _CHUNK), jnp.int32),
          pltpu.VMEM((_CHUNK, 16), jnp.float32),
          pltpu.VMEM((_CHUNK, 16), jnp.float32),
          pltpu.VMEM_SHARED((_N_PAD, 16), jnp.float32),
      ],
  )
  def degk(dst_hbm, ones_hbm, z_hbm, out_hbm, dst_v, ones_v, z_v, acc):
    c = lax.axis_index("c")
    s = lax.axis_index("s")
    row0 = (c * _NT + s) * _CH_W
    pltpu.sync_copy(dst_hbm.at[pl.ds(row0, _CH_W)], dst_v)
    pltpu.sync_copy(ones_hbm, ones_v)
    pltpu.sync_copy(z_hbm, z_v)
    for k in range(_ZR // _CHUNK):
      pltpu.sync_copy(z_v, acc.at[pl.ds(s * _ZR + k * _CHUNK, _CHUNK)])
    plsc.subcore_barrier()

    def step(j, carry):
      pltpu.sync_copy(ones_v, acc.at[dst_v.at[j]], add=True)
      return carry

    lax.fori_loop(0, _CH_W, step, 0)
    plsc.subcore_barrier()
    for k in range(_ZR // _CHUNK):
      r = s * _ZR + k * _CHUNK
      pltpu.sync_copy(acc.at[pl.ds(r, _CHUNK)], out_hbm.at[c, pl.ds(r, _CHUNK)])

  return degk


_sc_degree = _make_sc_degree()


def _mm_body(x_ref, w_ref, o_ref):
  o_ref[...] = jnp.dot(x_ref[...], w_ref[...],
                       preferred_element_type=jnp.float32)


def _scale1_body(dp0_ref, dp1_ref, xw_ref, dinv_ref, y_ref):
  deg = dp0_ref[...][:, 0:1] + dp1_ref[...][:, 0:1] + 1.0
  dinv = lax.rsqrt(deg)
  dinv_ref[...] = dinv
  y_ref[...] = xw_ref[...] * dinv


def _layer2_body(a0_ref, a1_ref, xw_ref, dinv_ref, b1_ref, w2_ref, y2_ref):
  dinv = dinv_ref[...]
  h = dinv * (a0_ref[...] + a1_ref[...]) + (dinv * dinv) * xw_ref[...]
  h = jnp.maximum(h + b1_ref[...], 0.0)
  z = jnp.dot(h, w2_ref[...], preferred_element_type=jnp.float32)
  y2_ref[...] = dinv * z


def _final_body(a0_ref, a1_ref, y2_ref, dinv_ref, b2_ref, o_ref):
  o = dinv_ref[...] * (a0_ref[...] + a1_ref[...] + y2_ref[...]) + b2_ref[...]
  m = jnp.max(o, axis=1, keepdims=True)
  lse = jnp.log(jnp.sum(jnp.exp(o - m), axis=1, keepdims=True)) + m
  o_ref[...] = o - lse


def _rows(shape):
  return pl.BlockSpec(shape, lambda i: (i, 0))


def kernel(x, edge_index, W1, b1, W2, b2):
  src = edge_index[0].astype(jnp.int32)
  dst = edge_index[1].astype(jnp.int32)
  pad = _E_PAD - _E
  src2d = jnp.concatenate([src, jnp.zeros((pad,), jnp.int32)]).reshape(-1, _CHUNK)
  dst2d = jnp.concatenate([dst, jnp.full((pad,), _N, jnp.int32)]).reshape(-1, _CHUNK)

  z_h = jnp.zeros((_CHUNK, _H), jnp.float32)
  z_o = jnp.zeros((_CHUNK, _O), jnp.float32)
  z16 = jnp.zeros((_CHUNK, 16), jnp.float32)
  ones16 = jnp.ones((_CHUNK, 16), jnp.float32)

  grid = (_N // _RB,)

  degp = _sc_degree(dst2d, ones16, z16)

  xw1 = pl.pallas_call(
      _mm_body, grid=grid,
      in_specs=[_rows((_RB, _D)), pl.BlockSpec((_D, _H), lambda i: (0, 0))],
      out_specs=_rows((_RB, _H)),
      out_shape=jax.ShapeDtypeStruct((_N, _H), jnp.float32),
  )(x, W1)

  dinv, y1 = pl.pallas_call(
      _scale1_body, grid=grid,
      in_specs=[_rows((_RB, 16)), _rows((_RB, 16)), _rows((_RB, _H))],
      out_specs=(_rows((_RB, 1)), _rows((_RB, _H))),
      out_shape=(jax.ShapeDtypeStruct((_N, 1), jnp.float32),
                 jax.ShapeDtypeStruct((_N, _H), jnp.float32)),
  )(degp[0, :_N], degp[1, :_N], xw1)

  acc1 = _sc_scatter_h(y1, src2d, dst2d, z_h)

  y2 = pl.pallas_call(
      _layer2_body, grid=grid,
      in_specs=[_rows((_RB, _H)), _rows((_RB, _H)), _rows((_RB, _H)),
                _rows((_RB, 1)), pl.BlockSpec((1, _H), lambda i: (0, 0)),
                pl.BlockSpec((_H, _O), lambda i: (0, 0))],
      out_specs=_rows((_RB, _O)),
      out_shape=jax.ShapeDtypeStruct((_N, _O), jnp.float32),
  )(acc1[0, :_N], acc1[1, :_N], xw1, dinv, b1.reshape(1, _H), W2)

  acc2 = _sc_scatter_o(y2, src2d, dst2d, z_o)

  out = pl.pallas_call(
      _final_body, grid=grid,
      in_specs=[_rows((_RB, _O)), _rows((_RB, _O)), _rows((_RB, _O)),
                _rows((_RB, 1)), pl.BlockSpec((1, _O), lambda i: (0, 0))],
      out_specs=_rows((_RB, _O)),
      out_shape=jax.ShapeDtypeStruct((_N, _O), jnp.float32),
  )(acc2[0, :_N], acc2[1, :_N], y2, dinv, b2.reshape(1, _O))
  return out


# speedup vs baseline: 1.7371x; 1.5645x over previous
import functools

import jax
import jax.numpy as jnp
from jax import lax
from jax.experimental import pallas as pl
from jax.experimental.pallas import tpu as pltpu
from jax.experimental.pallas import tpu_sc as plsc

_N = 10000
_E = 320000
_D = 128
_H = 128
_O = 64

_NC = 2
_NT = 16
_NW = _NC * _NT

_CHUNK = 128
_N_PAD = 10240
_E_PAD = 327680
_CH_T = _E_PAD // (_NT * _CHUNK)
_SCH = 8
_NSCH = _CH_T // _SCH
_ZR = _N_PAD // _NT
_YR = _N // _NT

_RB = 2000


def _make_sc_scatter(half, nbuf=2):
  mesh = plsc.VectorSubcoreMesh(core_axis_name="c", subcore_axis_name="s")

  @functools.partial(
      pl.kernel,
      out_type=jax.ShapeDtypeStruct((_NC, _N_PAD, half), jnp.float32),
      mesh=mesh,
      compiler_params=pltpu.CompilerParams(use_tc_tiling_on_sc=False),
      scratch_types=[
          pltpu.VMEM((_SCH, _CHUNK), jnp.int32),
          pltpu.VMEM((_SCH, _CHUNK), jnp.int32),
          [pltpu.VMEM((_CHUNK, half), jnp.float32) for _ in range(nbuf)],
          pltpu.VMEM_SHARED((_N, half), jnp.float32),
          pltpu.VMEM_SHARED((_N_PAD, half), jnp.float32),
          [pltpu.SemaphoreType.DMA for _ in range(nbuf)],
      ],
  )
  def scat(y_hbm, src_hbm, dst_hbm, z_hbm, out_hbm,
           src_v, dst_v, bufs, y_v, acc, sems):
    c = lax.axis_index("c")
    s = lax.axis_index("s")
    with jax.named_scope("stage"):
      pltpu.sync_copy(y_hbm.at[c, pl.ds(s * _YR, _YR)], y_v.at[pl.ds(s * _YR, _YR)])
      pltpu.sync_copy(z_hbm, bufs[0])
      for k in range(_ZR // _CHUNK):
        pltpu.sync_copy(bufs[0], acc.at[pl.ds(s * _ZR + k * _CHUNK, _CHUNK)])
      plsc.subcore_barrier()

    def fire(j, buf, sem):
      pltpu.async_copy(y_v.at[src_v.at[j]], buf, sem)

    def outer(g, carry):
      base = s * _CH_T + g * _SCH
      pltpu.sync_copy(src_hbm.at[pl.ds(base, _SCH)], src_v)
      pltpu.sync_copy(dst_hbm.at[pl.ds(base, _SCH)], dst_v)
      for b in range(nbuf):
        fire(b, bufs[b], sems[b])

      def step(i, c2):
        j0 = i * nbuf
        for b in range(nbuf):
          j = j0 + b
          pltpu.make_async_copy(y_v.at[src_v.at[j]], bufs[b], sems[b]).wait()
          pltpu.sync_copy(bufs[b], acc.at[dst_v.at[j]], add=True)

          @pl.when(j + nbuf < _SCH)
          def _(b=b, j=j):
            fire(j + nbuf, bufs[b], sems[b])
        return c2

      lax.fori_loop(0, _SCH // nbuf, step, 0)
      return carry

    with jax.named_scope("edge_loop"):
      lax.fori_loop(0, _NSCH, outer, 0)
      plsc.subcore_barrier()
    with jax.named_scope("acc_writeout"):
      for k in range(_ZR // _CHUNK):
        r = s * _ZR + k * _CHUNK
        pltpu.sync_copy(acc.at[pl.ds(r, _CHUNK)], out_hbm.at[c, pl.ds(r, _CHUNK)])

  return scat


_sc_scatter_h = _make_sc_scatter(_H // 2, nbuf=2)
_sc_scatter_o = _make_sc_scatter(_O // 2, nbuf=4)


def _make_sc_degree():
  mesh = plsc.VectorSubcoreMesh(core_axis_name="c", subcore_axis_name="s")

  _CH_W = _E_PAD // (_NW * _CHUNK)

  @functools.partial(
      pl.kernel,
      out_type=jax.ShapeDtypeStruct((_NC, _N_PAD, 16), jnp.float32),
      mesh=mesh,
      compiler_params=pltpu.CompilerParams(use_tc_tiling_on_sc=False),
      scratch_types=[
          pltpu.VMEM((_E_PAD // (_NW * _CHUNK), _CHUNK), jnp.int32),
          pltpu.VMEM((_CHUNK, 16), jnp.float32),
          pltpu.VMEM((_CHUNK, 16), jnp.float32),
          pltpu.VMEM_SHARED((_N_PAD, 16), jnp.float32),
      ],
  )
  def degk(dst_hbm, ones_hbm, z_hbm, out_hbm, dst_v, ones_v, z_v, acc):
    c = lax.axis_index("c")
    s = lax.axis_index("s")
    row0 = (c * _NT + s) * _CH_W
    pltpu.sync_copy(dst_hbm.at[pl.ds(row0, _CH_W)], dst_v)
    pltpu.sync_copy(ones_hbm, ones_v)
    pltpu.sync_copy(z_hbm, z_v)
    for k in range(_ZR // _CHUNK):
      pltpu.sync_copy(z_v, acc.at[pl.ds(s * _ZR + k * _CHUNK, _CHUNK)])
    plsc.subcore_barrier()

    def step(j, carry):
      pltpu.sync_copy(ones_v, acc.at[dst_v.at[j]], add=True)
      return carry

    lax.fori_loop(0, _CH_W, step, 0)
    plsc.subcore_barrier()
    for k in range(_ZR // _CHUNK):
      r = s * _ZR + k * _CHUNK
      pltpu.sync_copy(acc.at[pl.ds(r, _CHUNK)], out_hbm.at[c, pl.ds(r, _CHUNK)])

  return degk


_sc_degree = _make_sc_degree()


def _mm_body(x_ref, w_ref, o_ref):
  o_ref[...] = jnp.dot(x_ref[...], w_ref[...],
                       preferred_element_type=jnp.float32)


def _scale1_body(dp0_ref, dp1_ref, xw_ref, dinv_ref, y_ref):
  deg = dp0_ref[...][:, 0:1] + dp1_ref[...][:, 0:1] + 1.0
  dinv = lax.rsqrt(deg)
  dinv_ref[...] = dinv
  y = xw_ref[...] * dinv
  y_ref[0] = y[:, : _H // 2]
  y_ref[1] = y[:, _H // 2 :]


def _layer2_body(a_ref, xw_ref, dinv_ref, b1_ref, w2_ref, y2_ref):
  dinv = dinv_ref[...]
  agg = jnp.concatenate([a_ref[0], a_ref[1]], axis=1)
  h = dinv * agg + (dinv * dinv) * xw_ref[...]
  h = jnp.maximum(h + b1_ref[...], 0.0)
  z = jnp.dot(h, w2_ref[...], preferred_element_type=jnp.float32)
  y2 = dinv * z
  y2_ref[0] = y2[:, : _O // 2]
  y2_ref[1] = y2[:, _O // 2 :]


def _final_body(a_ref, y2_ref, dinv_ref, b2_ref, o_ref):
  agg = jnp.concatenate([a_ref[0], a_ref[1]], axis=1)
  y2 = jnp.concatenate([y2_ref[0], y2_ref[1]], axis=1)
  o = dinv_ref[...] * (agg + y2) + b2_ref[...]
  m = jnp.max(o, axis=1, keepdims=True)
  lse = jnp.log(jnp.sum(jnp.exp(o - m), axis=1, keepdims=True)) + m
  o_ref[...] = o - lse


def _rows(shape):
  return pl.BlockSpec(shape, lambda i: (i, 0))


def _rows3(shape):
  return pl.BlockSpec(shape, lambda i: (0, i, 0))


def kernel(x, edge_index, W1, b1, W2, b2):
  src = edge_index[0].astype(jnp.int32)
  dst = edge_index[1].astype(jnp.int32)
  pad = _E_PAD - _E
  src2d = jnp.concatenate([src, jnp.zeros((pad,), jnp.int32)]).reshape(-1, _CHUNK)
  dst2d = jnp.concatenate([dst, jnp.full((pad,), _N, jnp.int32)]).reshape(-1, _CHUNK)

  z_h = jnp.zeros((_CHUNK, _H // 2), jnp.float32)
  z_o = jnp.zeros((_CHUNK, _O // 2), jnp.float32)
  z16 = jnp.zeros((_CHUNK, 16), jnp.float32)
  ones16 = jnp.ones((_CHUNK, 16), jnp.float32)

  grid = (_N // _RB,)

  degp = _sc_degree(dst2d, ones16, z16)

  xw1 = pl.pallas_call(
      _mm_body, grid=grid,
      in_specs=[_rows((_RB, _D)), pl.BlockSpec((_D, _H), lambda i: (0, 0))],
      out_specs=_rows((_RB, _H)),
      out_shape=jax.ShapeDtypeStruct((_N, _H), jnp.float32),
  )(x, W1)

  dinv, y1h = pl.pallas_call(
      _scale1_body, grid=grid,
      in_specs=[_rows((_RB, 16)), _rows((_RB, 16)), _rows((_RB, _H))],
      out_specs=(_rows((_RB, 1)), _rows3((_NC, _RB, _H // 2))),
      out_shape=(jax.ShapeDtypeStruct((_N, 1), jnp.float32),
                 jax.ShapeDtypeStruct((_NC, _N, _H // 2), jnp.float32)),
  )(degp[0, :_N], degp[1, :_N], xw1)

  acc1 = _sc_scatter_h(y1h, src2d, dst2d, z_h)

  y2h = pl.pallas_call(
      _layer2_body, grid=grid,
      in_specs=[_rows3((_NC, _RB, _H // 2)), _rows((_RB, _H)),
                _rows((_RB, 1)), pl.BlockSpec((1, _H), lambda i: (0, 0)),
                pl.BlockSpec((_H, _O), lambda i: (0, 0))],
      out_specs=_rows3((_NC, _RB, _O // 2)),
      out_shape=jax.ShapeDtypeStruct((_NC, _N, _O // 2), jnp.float32),
  )(acc1[:, :_N], xw1, dinv, b1.reshape(1, _H), W2)

  acc2 = _sc_scatter_o(y2h, src2d, dst2d, z_o)

  out = pl.pallas_call(
      _final_body, grid=grid,
      in_specs=[_rows3((_NC, _RB, _O // 2)), _rows3((_NC, _RB, _O // 2)),
                _rows((_RB, 1)), pl.BlockSpec((1, _O), lambda i: (0, 0))],
      out_specs=_rows((_RB, _O)),
      out_shape=jax.ShapeDtypeStruct((_N, _O), jnp.float32),
  )(acc2[:, :_N], y2h, dinv, b2.reshape(1, _O))
  return out


# speedup vs baseline: 1.7456x; 1.0049x over previous
import functools

import jax
import jax.numpy as jnp
from jax import lax
from jax.experimental import pallas as pl
from jax.experimental.pallas import tpu as pltpu
from jax.experimental.pallas import tpu_sc as plsc

_N = 10000
_E = 320000
_D = 128
_H = 128
_O = 64

_NC = 2
_NT = 16
_NW = _NC * _NT

_CHUNK = 128
_N_PAD = 10240
_E_PAD = 327680
_CH_T = _E_PAD // (_NT * _CHUNK)
_SCH = 8
_NSCH = _CH_T // _SCH
_ZR = _N_PAD // _NT
_YR = _N // _NT

_RB = 2000


def _make_sc_scatter(half, nbuf=2):
  mesh = plsc.VectorSubcoreMesh(core_axis_name="c", subcore_axis_name="s")

  @functools.partial(
      pl.kernel,
      out_type=jax.ShapeDtypeStruct((_NC, _N_PAD, half), jnp.float32),
      mesh=mesh,
      compiler_params=pltpu.CompilerParams(use_tc_tiling_on_sc=False),
      scratch_types=[
          pltpu.VMEM((_SCH, _CHUNK), jnp.int32),
          pltpu.VMEM((_SCH, _CHUNK), jnp.int32),
          [pltpu.VMEM((_CHUNK, half), jnp.float32) for _ in range(nbuf)],
          pltpu.VMEM_SHARED((_N, half), jnp.float32),
          pltpu.VMEM_SHARED((_N_PAD, half), jnp.float32),
          [pltpu.SemaphoreType.DMA for _ in range(nbuf)],
      ],
  )
  def scat(y_hbm, src_hbm, dst_hbm, z_hbm, out_hbm,
           src_v, dst_v, bufs, y_v, acc, sems):
    c = lax.axis_index("c")
    s = lax.axis_index("s")
    with jax.named_scope("stage"):
      pltpu.sync_copy(y_hbm.at[c, pl.ds(s * _YR, _YR)], y_v.at[pl.ds(s * _YR, _YR)])
      pltpu.sync_copy(z_hbm, bufs[0])
      for k in range(_ZR // _CHUNK):
        pltpu.sync_copy(bufs[0], acc.at[pl.ds(s * _ZR + k * _CHUNK, _CHUNK)])
      plsc.subcore_barrier()

    def fire(j, buf, sem):
      pltpu.async_copy(y_v.at[src_v.at[j]], buf, sem)

    def outer(g, carry):
      base = s * _CH_T + g * _SCH
      pltpu.sync_copy(src_hbm.at[pl.ds(base, _SCH)], src_v)
      pltpu.sync_copy(dst_hbm.at[pl.ds(base, _SCH)], dst_v)
      for b in range(nbuf):
        fire(b, bufs[b], sems[b])

      def step(i, c2):
        j0 = i * nbuf
        for b in range(nbuf):
          j = j0 + b
          pltpu.make_async_copy(y_v.at[src_v.at[j]], bufs[b], sems[b]).wait()
          pltpu.sync_copy(bufs[b], acc.at[dst_v.at[j]], add=True)

          @pl.when(j + nbuf < _SCH)
          def _(b=b, j=j):
            fire(j + nbuf, bufs[b], sems[b])
        return c2

      lax.fori_loop(0, _SCH // nbuf, step, 0)
      return carry

    with jax.named_scope("edge_loop"):
      lax.fori_loop(0, _NSCH, outer, 0)
      plsc.subcore_barrier()
    with jax.named_scope("acc_writeout"):
      for k in range(_ZR // _CHUNK):
        r = s * _ZR + k * _CHUNK
        pltpu.sync_copy(acc.at[pl.ds(r, _CHUNK)], out_hbm.at[c, pl.ds(r, _CHUNK)])

  return scat


_sc_scatter_h = _make_sc_scatter(_H // 2, nbuf=4)
_sc_scatter_o = _make_sc_scatter(_O // 2, nbuf=4)


def _make_sc_degree():
  mesh = plsc.VectorSubcoreMesh(core_axis_name="c", subcore_axis_name="s")

  _CH_W = _E_PAD // (_NW * _CHUNK)

  @functools.partial(
      pl.kernel,
      out_type=jax.ShapeDtypeStruct((_NC, _N_PAD, 16), jnp.float32),
      mesh=mesh,
      compiler_params=pltpu.CompilerParams(use_tc_tiling_on_sc=False),
      scratch_types=[
          pltpu.VMEM((_E_PAD // (_NW * _CHUNK), _CHUNK), jnp.int32),
          pltpu.VMEM((_CHUNK, 16), jnp.float32),
          pltpu.VMEM((_CHUNK, 16), jnp.float32),
          pltpu.VMEM_SHARED((_N_PAD, 16), jnp.float32),
      ],
  )
  def degk(dst_hbm, ones_hbm, z_hbm, out_hbm, dst_v, ones_v, z_v, acc):
    c = lax.axis_index("c")
    s = lax.axis_index("s")
    row0 = (c * _NT + s) * _CH_W
    pltpu.sync_copy(dst_hbm.at[pl.ds(row0, _CH_W)], dst_v)
    pltpu.sync_copy(ones_hbm, ones_v)
    pltpu.sync_copy(z_hbm, z_v)
    for k in range(_ZR // _CHUNK):
      pltpu.sync_copy(z_v, acc.at[pl.ds(s * _ZR + k * _CHUNK, _CHUNK)])
    plsc.subcore_barrier()

    def step(j, carry):
      pltpu.sync_copy(ones_v, acc.at[dst_v.at[j]], add=True)
      return carry

    lax.fori_loop(0, _CH_W, step, 0)
    plsc.subcore_barrier()
    for k in range(_ZR // _CHUNK):
      r = s * _ZR + k * _CHUNK
      pltpu.sync_copy(acc.at[pl.ds(r, _CHUNK)], out_hbm.at[c, pl.ds(r, _CHUNK)])

  return degk


_sc_degree = _make_sc_degree()


def _mm_body(x_ref, w_ref, o_ref):
  o_ref[...] = jnp.dot(x_ref[...], w_ref[...],
                       preferred_element_type=jnp.float32)


def _scale1_body(dp0_ref, dp1_ref, xw_ref, dinv_ref, y_ref):
  deg = dp0_ref[...][:, 0:1] + dp1_ref[...][:, 0:1] + 1.0
  dinv = lax.rsqrt(deg)
  dinv_ref[...] = dinv
  y = xw_ref[...] * dinv
  y_ref[0] = y[:, : _H // 2]
  y_ref[1] = y[:, _H // 2 :]


def _layer2_body(a_ref, xw_ref, dinv_ref, b1_ref, w2_ref, y2_ref):
  dinv = dinv_ref[...]
  agg = jnp.concatenate([a_ref[0], a_ref[1]], axis=1)
  h = dinv * agg + (dinv * dinv) * xw_ref[...]
  h = jnp.maximum(h + b1_ref[...], 0.0)
  z = jnp.dot(h, w2_ref[...], preferred_element_type=jnp.float32)
  y2 = dinv * z
  y2_ref[0] = y2[:, : _O // 2]
  y2_ref[1] = y2[:, _O // 2 :]


def _final_body(a_ref, y2_ref, dinv_ref, b2_ref, o_ref):
  agg = jnp.concatenate([a_ref[0], a_ref[1]], axis=1)
  y2 = jnp.concatenate([y2_ref[0], y2_ref[1]], axis=1)
  o = dinv_ref[...] * (agg + y2) + b2_ref[...]
  m = jnp.max(o, axis=1, keepdims=True)
  lse = jnp.log(jnp.sum(jnp.exp(o - m), axis=1, keepdims=True)) + m
  o_ref[...] = o - lse


def _rows(shape):
  return pl.BlockSpec(shape, lambda i: (i, 0))


def _rows3(shape):
  return pl.BlockSpec(shape, lambda i: (0, i, 0))


def kernel(x, edge_index, W1, b1, W2, b2):
  src = edge_index[0].astype(jnp.int32)
  dst = edge_index[1].astype(jnp.int32)
  pad = _E_PAD - _E
  src2d = jnp.concatenate([src, jnp.zeros((pad,), jnp.int32)]).reshape(-1, _CHUNK)
  dst2d = jnp.concatenate([dst, jnp.full((pad,), _N, jnp.int32)]).reshape(-1, _CHUNK)

  z_h = jnp.zeros((_CHUNK, _H // 2), jnp.float32)
  z_o = jnp.zeros((_CHUNK, _O // 2), jnp.float32)
  z16 = jnp.zeros((_CHUNK, 16), jnp.float32)
  ones16 = jnp.ones((_CHUNK, 16), jnp.float32)

  grid = (_N // _RB,)

  degp = _sc_degree(dst2d, ones16, z16)

  xw1 = pl.pallas_call(
      _mm_body, grid=grid,
      in_specs=[_rows((_RB, _D)), pl.BlockSpec((_D, _H), lambda i: (0, 0))],
      out_specs=_rows((_RB, _H)),
      out_shape=jax.ShapeDtypeStruct((_N, _H), jnp.float32),
  )(x, W1)

  dinv, y1h = pl.pallas_call(
      _scale1_body, grid=grid,
      in_specs=[_rows((_RB, 16)), _rows((_RB, 16)), _rows((_RB, _H))],
      out_specs=(_rows((_RB, 1)), _rows3((_NC, _RB, _H // 2))),
      out_shape=(jax.ShapeDtypeStruct((_N, 1), jnp.float32),
                 jax.ShapeDtypeStruct((_NC, _N, _H // 2), jnp.float32)),
  )(degp[0, :_N], degp[1, :_N], xw1)

  acc1 = _sc_scatter_h(y1h, src2d, dst2d, z_h)

  y2h = pl.pallas_call(
      _layer2_body, grid=grid,
      in_specs=[_rows3((_NC, _RB, _H // 2)), _rows((_RB, _H)),
                _rows((_RB, 1)), pl.BlockSpec((1, _H), lambda i: (0, 0)),
                pl.BlockSpec((_H, _O), lambda i: (0, 0))],
      out_specs=_rows3((_NC, _RB, _O // 2)),
      out_shape=jax.ShapeDtypeStruct((_NC, _N, _O // 2), jnp.float32),
  )(acc1[:, :_N], xw1, dinv, b1.reshape(1, _H), W2)

  acc2 = _sc_scatter_o(y2h, src2d, dst2d, z_o)

  out = pl.pallas_call(
      _final_body, grid=grid,
      in_specs=[_rows3((_NC, _RB, _O // 2)), _rows3((_NC, _RB, _O // 2)),
                _rows((_RB, 1)), pl.BlockSpec((1, _O), lambda i: (0, 0))],
      out_specs=_rows((_RB, _O)),
      out_shape=jax.ShapeDtypeStruct((_N, _O), jnp.float32),
  )(acc2[:, :_N], y2h, dinv, b2.reshape(1, _O))
  return out


# speedup vs baseline: 2.0303x; 1.1631x over previous
import functools

import jax
import jax.numpy as jnp
from jax import lax
from jax.experimental import pallas as pl
from jax.experimental.pallas import tpu as pltpu
from jax.experimental.pallas import tpu_sc as plsc

_N = 10000
_E = 320000
_D = 128
_H = 128
_O = 64

_NC = 2
_NT = 16
_NW = _NC * _NT

_CHUNK = 128
_N_PAD = 10240
_E_PAD = 327680
_CH_T = _E_PAD // (_NT * _CHUNK)
_SCH = 8
_NSCH = _CH_T // _SCH
_ZR = _N_PAD // _NT
_YR = _N // _NT
_WR = _YR // 5

_RB = 2000


def _make_sc_scatter(half, nbuf):
  mesh = plsc.VectorSubcoreMesh(core_axis_name="c", subcore_axis_name="s")

  @functools.partial(
      pl.kernel,
      out_type=jax.ShapeDtypeStruct((_N, 128), jnp.float32),
      mesh=mesh,
      compiler_params=pltpu.CompilerParams(use_tc_tiling_on_sc=False),
      scratch_types=[
          pltpu.VMEM((_SCH, _CHUNK), jnp.int32),
          pltpu.VMEM((_SCH, _CHUNK), jnp.int32),
          [pltpu.VMEM((_CHUNK, half), jnp.float32) for _ in range(nbuf)],
          pltpu.VMEM_SHARED((_N, half), jnp.float32),
          pltpu.VMEM_SHARED((_N_PAD, half), jnp.float32),
          [pltpu.SemaphoreType.DMA for _ in range(nbuf)],
      ],
  )
  def scat(y_hbm, src_hbm, dst_hbm, z_hbm, out_hbm,
           src_v, dst_v, bufs, y_v, acc, sems):
    c = lax.axis_index("c")
    s = lax.axis_index("s")
    with jax.named_scope("stage"):
      for cc in range(_NC):
        @pl.when(c == cc)
        def _(cc=cc):
          pltpu.sync_copy(
              y_hbm.at[pl.ds(s * _YR, _YR), pl.ds(cc * half, half)],
              y_v.at[pl.ds(s * _YR, _YR)])
      pltpu.sync_copy(z_hbm, bufs[0])
      for k in range(_ZR // _CHUNK):
        pltpu.sync_copy(bufs[0], acc.at[pl.ds(s * _ZR + k * _CHUNK, _CHUNK)])
      plsc.subcore_barrier()

    def fire(j, buf, sem):
      pltpu.async_copy(y_v.at[src_v.at[j]], buf, sem)

    def outer(g, carry):
      base = s * _CH_T + g * _SCH
      pltpu.sync_copy(src_hbm.at[pl.ds(base, _SCH)], src_v)
      pltpu.sync_copy(dst_hbm.at[pl.ds(base, _SCH)], dst_v)
      for b in range(nbuf):
        fire(b, bufs[b], sems[b])

      def step(i, c2):
        j0 = i * nbuf
        for b in range(nbuf):
          j = j0 + b
          pltpu.make_async_copy(y_v.at[src_v.at[j]], bufs[b], sems[b]).wait()
          pltpu.sync_copy(bufs[b], acc.at[dst_v.at[j]], add=True)

          @pl.when(j + nbuf < _SCH)
          def _(b=b, j=j):
            fire(j + nbuf, bufs[b], sems[b])
        return c2

      lax.fori_loop(0, _SCH // nbuf, step, 0)
      return carry

    with jax.named_scope("edge_loop"):
      lax.fori_loop(0, _NSCH, outer, 0)
      plsc.subcore_barrier()
    with jax.named_scope("acc_writeout"):
      for k in range(_YR // _WR):
        r = s * _YR + k * _WR
        for cc in range(_NC):
          @pl.when(c == cc)
          def _(cc=cc, r=r):
            pltpu.sync_copy(acc.at[pl.ds(r, _WR)],
                            out_hbm.at[pl.ds(r, _WR), pl.ds(cc * half, half)])

  return scat


_sc_scatter_h = _make_sc_scatter(_H // 2, nbuf=4)
_sc_scatter_o = _make_sc_scatter(_O // 2, nbuf=4)


def _make_sc_degree():
  mesh = plsc.VectorSubcoreMesh(core_axis_name="c", subcore_axis_name="s")

  _CH_W = _E_PAD // (_NW * _CHUNK)

  @functools.partial(
      pl.kernel,
      out_type=jax.ShapeDtypeStruct((_N, 128), jnp.float32),
      mesh=mesh,
      compiler_params=pltpu.CompilerParams(use_tc_tiling_on_sc=False),
      scratch_types=[
          pltpu.VMEM((_E_PAD // (_NW * _CHUNK), _CHUNK), jnp.int32),
          pltpu.VMEM((_CHUNK, 16), jnp.float32),
          pltpu.VMEM((_CHUNK, 16), jnp.float32),
          pltpu.VMEM_SHARED((_N_PAD, 16), jnp.float32),
      ],
  )
  def degk(dst_hbm, ones_hbm, z_hbm, out_hbm, dst_v, ones_v, z_v, acc):
    c = lax.axis_index("c")
    s = lax.axis_index("s")
    row0 = (c * _NT + s) * _CH_W
    pltpu.sync_copy(dst_hbm.at[pl.ds(row0, _CH_W)], dst_v)
    pltpu.sync_copy(ones_hbm, ones_v)
    pltpu.sync_copy(z_hbm, z_v)
    for k in range(_ZR // _CHUNK):
      pltpu.sync_copy(z_v, acc.at[pl.ds(s * _ZR + k * _CHUNK, _CHUNK)])
    plsc.subcore_barrier()

    def step(j, carry):
      pltpu.sync_copy(ones_v, acc.at[dst_v.at[j]], add=True)
      return carry

    lax.fori_loop(0, _CH_W, step, 0)
    plsc.subcore_barrier()
    for k in range(_YR // _WR):
      r = s * _YR + k * _WR
      for cc in range(_NC):
        @pl.when(c == cc)
        def _(cc=cc, r=r):
          pltpu.sync_copy(acc.at[pl.ds(r, _WR)],
                          out_hbm.at[pl.ds(r, _WR), pl.ds(cc * 16, 16)])

  return degk


_sc_degree = _make_sc_degree()


def _mm_body(x_ref, w_ref, o_ref):
  o_ref[...] = jnp.dot(x_ref[...], w_ref[...],
                       preferred_element_type=jnp.float32)


def _scale1_body(dp_ref, xw_ref, dinv_ref, y_ref):
  dp = dp_ref[...]
  deg = dp[:, 0:1] + dp[:, 16:17] + 1.0
  dinv = lax.rsqrt(deg)
  dinv_ref[...] = dinv
  y_ref[...] = xw_ref[...] * dinv


def _layer2_body(a_ref, xw_ref, dinv_ref, b1_ref, w2_ref, y2_ref):
  dinv = dinv_ref[...]
  h = dinv * a_ref[...] + (dinv * dinv) * xw_ref[...]
  h = jnp.maximum(h + b1_ref[...], 0.0)
  z = jnp.dot(h, w2_ref[...], preferred_element_type=jnp.float32)
  y2_ref[...] = jnp.concatenate([dinv * z, jnp.zeros_like(z)], axis=1)


def _final_body(a_ref, y2_ref, dinv_ref, b2_ref, o_ref):
  o = dinv_ref[...] * (a_ref[...][:, :_O] + y2_ref[...][:, :_O]) + b2_ref[...]
  m = jnp.max(o, axis=1, keepdims=True)
  lse = jnp.log(jnp.sum(jnp.exp(o - m), axis=1, keepdims=True)) + m
  o_ref[...] = o - lse


def _rows(shape):
  return pl.BlockSpec(shape, lambda i: (i, 0))


def kernel(x, edge_index, W1, b1, W2, b2):
  src = edge_index[0].astype(jnp.int32)
  dst = edge_index[1].astype(jnp.int32)
  pad = _E_PAD - _E
  src2d = jnp.concatenate([src, jnp.zeros((pad,), jnp.int32)]).reshape(-1, _CHUNK)
  dst2d = jnp.concatenate([dst, jnp.full((pad,), _N, jnp.int32)]).reshape(-1, _CHUNK)

  z_h = jnp.zeros((_CHUNK, _H // 2), jnp.float32)
  z_o = jnp.zeros((_CHUNK, _O // 2), jnp.float32)
  z16 = jnp.zeros((_CHUNK, 16), jnp.float32)
  ones16 = jnp.ones((_CHUNK, 16), jnp.float32)

  grid = (_N // _RB,)

  degp = _sc_degree(dst2d, ones16, z16)

  xw1 = pl.pallas_call(
      _mm_body, grid=grid,
      in_specs=[_rows((_RB, _D)), pl.BlockSpec((_D, _H), lambda i: (0, 0))],
      out_specs=_rows((_RB, _H)),
      out_shape=jax.ShapeDtypeStruct((_N, _H), jnp.float32),
  )(x, W1)

  dinv, y1 = pl.pallas_call(
      _scale1_body, grid=grid,
      in_specs=[_rows((_RB, 128)), _rows((_RB, _H))],
      out_specs=(_rows((_RB, 1)), _rows((_RB, _H))),
      out_shape=(jax.ShapeDtypeStruct((_N, 1), jnp.float32),
                 jax.ShapeDtypeStruct((_N, _H), jnp.float32)),
  )(degp, xw1)

  acc1 = _sc_scatter_h(y1, src2d, dst2d, z_h)

  y2buf = pl.pallas_call(
      _layer2_body, grid=grid,
      in_specs=[_rows((_RB, _H)), _rows((_RB, _H)),
                _rows((_RB, 1)), pl.BlockSpec((1, _H), lambda i: (0, 0)),
                pl.BlockSpec((_H, _O), lambda i: (0, 0))],
      out_specs=_rows((_RB, 128)),
      out_shape=jax.ShapeDtypeStruct((_N, 128), jnp.float32),
  )(acc1, xw1, dinv, b1.reshape(1, _H), W2)

  acc2 = _sc_scatter_o(y2buf, src2d, dst2d, z_o)

  out = pl.pallas_call(
      _final_body, grid=grid,
      in_specs=[_rows((_RB, 128)), _rows((_RB, 128)),
                _rows((_RB, 1)), pl.BlockSpec((1, _O), lambda i: (0, 0))],
      out_specs=_rows((_RB, _O)),
      out_shape=jax.ShapeDtypeStruct((_N, _O), jnp.float32),
  )(acc2, y2buf, dinv, b2.reshape(1, _O))
  return out


# speedup vs baseline: 2.0317x; 1.0007x over previous
import functools

import jax
import jax.numpy as jnp
from jax import lax
from jax.experimental import pallas as pl
from jax.experimental.pallas import tpu as pltpu
from jax.experimental.pallas import tpu_sc as plsc

_N = 10000
_E = 320000
_D = 128
_H = 128
_O = 64

_NC = 2
_NT = 16
_NW = _NC * _NT

_CHUNK = 128
_N_PAD = 10240
_E_PAD = 327680
_CH_T = _E_PAD // (_NT * _CHUNK)
_SCH = 8
_NSCH = _CH_T // _SCH
_ZR = _N_PAD // _NT
_YR = _N // _NT
_WR = _YR // 5

_RB = 2000


def _make_sc_scatter(half, nbuf):
  mesh = plsc.VectorSubcoreMesh(core_axis_name="c", subcore_axis_name="s")

  @functools.partial(
      pl.kernel,
      out_type=jax.ShapeDtypeStruct((_N, 128), jnp.float32),
      mesh=mesh,
      compiler_params=pltpu.CompilerParams(use_tc_tiling_on_sc=False),
      scratch_types=[
          pltpu.VMEM((2, _SCH, _CHUNK), jnp.int32),
          pltpu.VMEM((2, _SCH, _CHUNK), jnp.int32),
          [pltpu.VMEM((_CHUNK, half), jnp.float32) for _ in range(nbuf)],
          pltpu.VMEM_SHARED((_N, half), jnp.float32),
          pltpu.VMEM_SHARED((_N_PAD, half), jnp.float32),
          [pltpu.SemaphoreType.DMA for _ in range(nbuf)],
          [pltpu.SemaphoreType.DMA for _ in range(nbuf)],
      ],
  )
  def scat(y_hbm, src_hbm, dst_hbm, z_hbm, out_hbm,
           src_v, dst_v, bufs, y_v, acc, gsems, ssems):
    c = lax.axis_index("c")
    s = lax.axis_index("s")
    with jax.named_scope("stage"):
      for cc in range(_NC):
        @pl.when(c == cc)
        def _(cc=cc):
          pltpu.sync_copy(
              y_hbm.at[pl.ds(s * _YR, _YR), pl.ds(cc * half, half)],
              y_v.at[pl.ds(s * _YR, _YR)])
      pltpu.sync_copy(z_hbm, bufs[0])
      for k in range(_ZR // _CHUNK):
        pltpu.sync_copy(bufs[0], acc.at[pl.ds(s * _ZR + k * _CHUNK, _CHUNK)])
      pltpu.sync_copy(src_hbm.at[pl.ds(s * _CH_T, _SCH)], src_v.at[0])
      pltpu.sync_copy(dst_hbm.at[pl.ds(s * _CH_T, _SCH)], dst_v.at[0])
      plsc.subcore_barrier()

    def g_fire(p, j, b):
      pltpu.async_copy(y_v.at[src_v.at[p, j]], bufs[b], gsems[b])

    def g_wait(p, j, b):
      pltpu.make_async_copy(y_v.at[src_v.at[p, j]], bufs[b], gsems[b]).wait()

    def s_fire(p, j, b):
      pltpu.async_copy(bufs[b], acc.at[dst_v.at[p, j]], ssems[b], add=True)

    def s_wait(p, j, b):
      pltpu.make_async_copy(bufs[b], acc.at[dst_v.at[p, j]], ssems[b]).wait()

    def outer(g, carry):
      p = lax.rem(g, 2)
      for b in range(nbuf):
        @pl.when(g > 0)
        def _(b=b):
          s_wait(1 - p, nbuf + b, b)
        g_fire(p, b, b)
      @pl.when(g + 1 < _NSCH)
      def _():
        base = s * _CH_T + (g + 1) * _SCH
        pltpu.sync_copy(src_hbm.at[pl.ds(base, _SCH)], src_v.at[1 - p])
        pltpu.sync_copy(dst_hbm.at[pl.ds(base, _SCH)], dst_v.at[1 - p])
      for b in range(nbuf):
        g_wait(p, b, b)
        s_fire(p, b, b)
      for b in range(nbuf):
        s_wait(p, b, b)
        g_fire(p, nbuf + b, b)
      for b in range(nbuf):
        g_wait(p, nbuf + b, b)
        s_fire(p, nbuf + b, b)
      return carry

    with jax.named_scope("edge_loop"):
      lax.fori_loop(0, _NSCH, outer, 0)
      p_last = lax.rem(_NSCH - 1, 2)
      for b in range(nbuf):
        s_wait(p_last, nbuf + b, b)
      plsc.subcore_barrier()
    with jax.named_scope("acc_writeout"):
      for k in range(_YR // _WR):
        r = s * _YR + k * _WR
        for cc in range(_NC):
          @pl.when(c == cc)
          def _(cc=cc, r=r):
            pltpu.sync_copy(acc.at[pl.ds(r, _WR)],
                            out_hbm.at[pl.ds(r, _WR), pl.ds(cc * half, half)])

  return scat


_sc_scatter_h = _make_sc_scatter(_H // 2, nbuf=4)
_sc_scatter_o = _make_sc_scatter(_O // 2, nbuf=4)


def _make_sc_degree():
  mesh = plsc.VectorSubcoreMesh(core_axis_name="c", subcore_axis_name="s")

  _CH_W = _E_PAD // (_NW * _CHUNK)

  @functools.partial(
      pl.kernel,
      out_type=jax.ShapeDtypeStruct((_N, 128), jnp.float32),
      mesh=mesh,
      compiler_params=pltpu.CompilerParams(use_tc_tiling_on_sc=False),
      scratch_types=[
          pltpu.VMEM((_E_PAD // (_NW * _CHUNK), _CHUNK), jnp.int32),
          pltpu.VMEM((_CHUNK, 16), jnp.float32),
          pltpu.VMEM((_CHUNK, 16), jnp.float32),
          pltpu.VMEM_SHARED((_N_PAD, 16), jnp.float32),
      ],
  )
  def degk(dst_hbm, ones_hbm, z_hbm, out_hbm, dst_v, ones_v, z_v, acc):
    c = lax.axis_index("c")
    s = lax.axis_index("s")
    row0 = (c * _NT + s) * _CH_W
    pltpu.sync_copy(dst_hbm.at[pl.ds(row0, _CH_W)], dst_v)
    pltpu.sync_copy(ones_hbm, ones_v)
    pltpu.sync_copy(z_hbm, z_v)
    for k in range(_ZR // _CHUNK):
      pltpu.sync_copy(z_v, acc.at[pl.ds(s * _ZR + k * _CHUNK, _CHUNK)])
    plsc.subcore_barrier()

    def step(j, carry):
      pltpu.sync_copy(ones_v, acc.at[dst_v.at[j]], add=True)
      return carry

    lax.fori_loop(0, _CH_W, step, 0)
    plsc.subcore_barrier()
    for k in range(_YR // _WR):
      r = s * _YR + k * _WR
      for cc in range(_NC):
        @pl.when(c == cc)
        def _(cc=cc, r=r):
          pltpu.sync_copy(acc.at[pl.ds(r, _WR)],
                          out_hbm.at[pl.ds(r, _WR), pl.ds(cc * 16, 16)])

  return degk


_sc_degree = _make_sc_degree()


def _mm_body(x_ref, w_ref, o_ref):
  o_ref[...] = jnp.dot(x_ref[...], w_ref[...],
                       preferred_element_type=jnp.float32)


def _scale1_body(dp_ref, xw_ref, dinv_ref, y_ref):
  dp = dp_ref[...]
  deg = dp[:, 0:1] + dp[:, 16:17] + 1.0
  dinv = lax.rsqrt(deg)
  dinv_ref[...] = dinv
  y_ref[...] = xw_ref[...] * dinv


def _layer2_body(a_ref, xw_ref, dinv_ref, b1_ref, w2_ref, y2_ref):
  dinv = dinv_ref[...]
  h = dinv * a_ref[...] + (dinv * dinv) * xw_ref[...]
  h = jnp.maximum(h + b1_ref[...], 0.0)
  z = jnp.dot(h, w2_ref[...], preferred_element_type=jnp.float32)
  y2_ref[...] = jnp.concatenate([dinv * z, jnp.zeros_like(z)], axis=1)


def _final_body(a_ref, y2_ref, dinv_ref, b2_ref, o_ref):
  o = dinv_ref[...] * (a_ref[...][:, :_O] + y2_ref[...][:, :_O]) + b2_ref[...]
  m = jnp.max(o, axis=1, keepdims=True)
  lse = jnp.log(jnp.sum(jnp.exp(o - m), axis=1, keepdims=True)) + m
  o_ref[...] = o - lse


def _rows(shape):
  return pl.BlockSpec(shape, lambda i: (i, 0))


def kernel(x, edge_index, W1, b1, W2, b2):
  src = edge_index[0].astype(jnp.int32)
  dst = edge_index[1].astype(jnp.int32)
  pad = _E_PAD - _E
  src2d = jnp.concatenate([src, jnp.zeros((pad,), jnp.int32)]).reshape(-1, _CHUNK)
  dst2d = jnp.concatenate([dst, jnp.full((pad,), _N, jnp.int32)]).reshape(-1, _CHUNK)

  z_h = jnp.zeros((_CHUNK, _H // 2), jnp.float32)
  z_o = jnp.zeros((_CHUNK, _O // 2), jnp.float32)
  z16 = jnp.zeros((_CHUNK, 16), jnp.float32)
  ones16 = jnp.ones((_CHUNK, 16), jnp.float32)

  grid = (_N // _RB,)

  degp = _sc_degree(dst2d, ones16, z16)

  xw1 = pl.pallas_call(
      _mm_body, grid=grid,
      in_specs=[_rows((_RB, _D)), pl.BlockSpec((_D, _H), lambda i: (0, 0))],
      out_specs=_rows((_RB, _H)),
      out_shape=jax.ShapeDtypeStruct((_N, _H), jnp.float32),
  )(x, W1)

  dinv, y1 = pl.pallas_call(
      _scale1_body, grid=grid,
      in_specs=[_rows((_RB, 128)), _rows((_RB, _H))],
      out_specs=(_rows((_RB, 1)), _rows((_RB, _H))),
      out_shape=(jax.ShapeDtypeStruct((_N, 1), jnp.float32),
                 jax.ShapeDtypeStruct((_N, _H), jnp.float32)),
  )(degp, xw1)

  acc1 = _sc_scatter_h(y1, src2d, dst2d, z_h)

  y2buf = pl.pallas_call(
      _layer2_body, grid=grid,
      in_specs=[_rows((_RB, _H)), _rows((_RB, _H)),
                _rows((_RB, 1)), pl.BlockSpec((1, _H), lambda i: (0, 0)),
                pl.BlockSpec((_H, _O), lambda i: (0, 0))],
      out_specs=_rows((_RB, 128)),
      out_shape=jax.ShapeDtypeStruct((_N, 128), jnp.float32),
  )(acc1, xw1, dinv, b1.reshape(1, _H), W2)

  acc2 = _sc_scatter_o(y2buf, src2d, dst2d, z_o)

  out = pl.pallas_call(
      _final_body, grid=grid,
      in_specs=[_rows((_RB, 128)), _rows((_RB, 128)),
                _rows((_RB, 1)), pl.BlockSpec((1, _O), lambda i: (0, 0))],
      out_specs=_rows((_RB, _O)),
      out_shape=jax.ShapeDtypeStruct((_N, _O), jnp.float32),
  )(acc2, y2buf, dinv, b2.reshape(1, _O))
  return out


# speedup vs baseline: 2.0403x; 1.0042x over previous
import functools

import jax
import jax.numpy as jnp
from jax import lax
from jax.experimental import pallas as pl
from jax.experimental.pallas import tpu as pltpu
from jax.experimental.pallas import tpu_sc as plsc

_N = 10000
_E = 320000
_D = 128
_H = 128
_O = 64

_NC = 2
_NT = 16
_NW = _NC * _NT

_CHUNK = 128
_N_PAD = 10240
_E_PAD = 327680
_CH_T = _E_PAD // (_NT * _CHUNK)
_SCH = 8
_NSCH = _CH_T // _SCH
_ZR = _N_PAD // _NT
_YR = _N // _NT
_WR = _YR // 5

_RB = 2000


def _make_sc_scatter(half, nbuf, kh0, kh1):
  mesh = plsc.VectorSubcoreMesh(core_axis_name="c", subcore_axis_name="s")

  @functools.partial(
      pl.kernel,
      out_type=(jax.ShapeDtypeStruct((_N, 128), jnp.float32),
                jax.ShapeDtypeStruct((_N, half), jnp.float32),
                jax.ShapeDtypeStruct((_N, half), jnp.float32)),
      mesh=mesh,
      compiler_params=pltpu.CompilerParams(use_tc_tiling_on_sc=False),
      scratch_types=[
          pltpu.VMEM((2, _SCH, _CHUNK), jnp.int32),
          pltpu.VMEM((2, _SCH, _CHUNK), jnp.int32),
          [pltpu.VMEM((_CHUNK, half), jnp.float32) for _ in range(nbuf)],
          pltpu.VMEM_SHARED((_N, half), jnp.float32),
          pltpu.VMEM_SHARED((_N_PAD, half), jnp.float32),
          [pltpu.SemaphoreType.DMA for _ in range(nbuf)],
          [pltpu.SemaphoreType.DMA for _ in range(nbuf)],
      ],
  )
  def scat(y_hbm, src_hbm, dst_hbm, z_hbm, out_hbm, yh0_hbm, yh1_hbm,
           src_v, dst_v, bufs, y_v, acc, gsems, ssems):
    c = lax.axis_index("c")
    s = lax.axis_index("s")
    yh = (yh0_hbm, yh1_hbm)
    with jax.named_scope("stage"):
      for cc in range(_NC):
        @pl.when(c == cc)
        def _(cc=cc):
          pltpu.sync_copy(
              y_hbm.at[pl.ds(s * _YR, _YR), pl.ds(cc * half, half)],
              y_v.at[pl.ds(s * _YR, _YR)])
          pltpu.sync_copy(y_v.at[pl.ds(s * _YR, _YR)],
                          yh[cc].at[pl.ds(s * _YR, _YR)])
      pltpu.sync_copy(z_hbm, bufs[0])
      for k in range(_ZR // _CHUNK):
        pltpu.sync_copy(bufs[0], acc.at[pl.ds(s * _ZR + k * _CHUNK, _CHUNK)])
      pltpu.sync_copy(src_hbm.at[pl.ds(s * _CH_T, _SCH)], src_v.at[0])
      pltpu.sync_copy(dst_hbm.at[pl.ds(s * _CH_T, _SCH)], dst_v.at[0])
      plsc.subcore_barrier()

    kh = jnp.where(c == 0, kh0, kh1)

    def g_fire(p, j, b, use_hbm):
      @pl.when(use_hbm)
      def _():
        for cc in range(_NC):
          @pl.when(c == cc)
          def _(cc=cc):
            pltpu.async_copy(yh[cc].at[src_v.at[p, j]], bufs[b], gsems[b])
      @pl.when(jnp.logical_not(use_hbm))
      def _():
        pltpu.async_copy(y_v.at[src_v.at[p, j]], bufs[b], gsems[b])

    def g_wait(p, j, b):
      pltpu.make_async_copy(y_v.at[src_v.at[p, j]], bufs[b], gsems[b]).wait()

    def s_fire(p, j, b):
      pltpu.async_copy(bufs[b], acc.at[dst_v.at[p, j]], ssems[b], add=True)

    def s_wait(p, j, b):
      pltpu.make_async_copy(bufs[b], acc.at[dst_v.at[p, j]], ssems[b]).wait()

    def outer(g, carry):
      p = lax.rem(g, 2)
      use_hbm = g < kh
      for b in range(nbuf):
        @pl.when(g > 0)
        def _(b=b):
          s_wait(1 - p, nbuf + b, b)
        g_fire(p, b, b, use_hbm)
      @pl.when(g + 1 < _NSCH)
      def _():
        base = s * _CH_T + (g + 1) * _SCH
        pltpu.sync_copy(src_hbm.at[pl.ds(base, _SCH)], src_v.at[1 - p])
        pltpu.sync_copy(dst_hbm.at[pl.ds(base, _SCH)], dst_v.at[1 - p])
      for b in range(nbuf):
        g_wait(p, b, b)
        s_fire(p, b, b)
      for b in range(nbuf):
        s_wait(p, b, b)
        g_fire(p, nbuf + b, b, use_hbm)
      for b in range(nbuf):
        g_wait(p, nbuf + b, b)
        s_fire(p, nbuf + b, b)
      return carry

    with jax.named_scope("edge_loop"):
      lax.fori_loop(0, _NSCH, outer, 0)
      p_last = lax.rem(_NSCH - 1, 2)
      for b in range(nbuf):
        s_wait(p_last, nbuf + b, b)
      plsc.subcore_barrier()
    with jax.named_scope("acc_writeout"):
      for k in range(_YR // _WR):
        r = s * _YR + k * _WR
        for cc in range(_NC):
          @pl.when(c == cc)
          def _(cc=cc, r=r):
            pltpu.sync_copy(acc.at[pl.ds(r, _WR)],
                            out_hbm.at[pl.ds(r, _WR), pl.ds(cc * half, half)])

  return scat


_sc_scatter_h = _make_sc_scatter(_H // 2, nbuf=4, kh0=6, kh1=2)
_sc_scatter_o = _make_sc_scatter(_O // 2, nbuf=4, kh0=6, kh1=2)


def _make_sc_degree():
  mesh = plsc.VectorSubcoreMesh(core_axis_name="c", subcore_axis_name="s")

  _CH_W = _E_PAD // (_NW * _CHUNK)

  @functools.partial(
      pl.kernel,
      out_type=jax.ShapeDtypeStruct((_N, 128), jnp.float32),
      mesh=mesh,
      compiler_params=pltpu.CompilerParams(use_tc_tiling_on_sc=False),
      scratch_types=[
          pltpu.VMEM((_E_PAD // (_NW * _CHUNK), _CHUNK), jnp.int32),
          pltpu.VMEM((_CHUNK, 16), jnp.float32),
          pltpu.VMEM((_CHUNK, 16), jnp.float32),
          pltpu.VMEM_SHARED((_N_PAD, 16), jnp.float32),
      ],
  )
  def degk(dst_hbm, ones_hbm, z_hbm, out_hbm, dst_v, ones_v, z_v, acc):
    c = lax.axis_index("c")
    s = lax.axis_index("s")
    row0 = (c * _NT + s) * _CH_W
    pltpu.sync_copy(dst_hbm.at[pl.ds(row0, _CH_W)], dst_v)
    pltpu.sync_copy(ones_hbm, ones_v)
    pltpu.sync_copy(z_hbm, z_v)
    for k in range(_ZR // _CHUNK):
      pltpu.sync_copy(z_v, acc.at[pl.ds(s * _ZR + k * _CHUNK, _CHUNK)])
    plsc.subcore_barrier()

    def step(j, carry):
      pltpu.sync_copy(ones_v, acc.at[dst_v.at[j]], add=True)
      return carry

    lax.fori_loop(0, _CH_W, step, 0)
    plsc.subcore_barrier()
    for k in range(_YR // _WR):
      r = s * _YR + k * _WR
      for cc in range(_NC):
        @pl.when(c == cc)
        def _(cc=cc, r=r):
          pltpu.sync_copy(acc.at[pl.ds(r, _WR)],
                          out_hbm.at[pl.ds(r, _WR), pl.ds(cc * 16, 16)])

  return degk


_sc_degree = _make_sc_degree()


def _mm_body(x_ref, w_ref, o_ref):
  o_ref[...] = jnp.dot(x_ref[...], w_ref[...],
                       preferred_element_type=jnp.float32)


def _scale1_body(dp_ref, xw_ref, dinv_ref, y_ref):
  dp = dp_ref[...]
  deg = dp[:, 0:1] + dp[:, 16:17] + 1.0
  dinv = lax.rsqrt(deg)
  dinv_ref[...] = dinv
  y_ref[...] = xw_ref[...] * dinv


def _layer2_body(a_ref, xw_ref, dinv_ref, b1_ref, w2_ref, y2_ref):
  dinv = dinv_ref[...]
  h = dinv * a_ref[...] + (dinv * dinv) * xw_ref[...]
  h = jnp.maximum(h + b1_ref[...], 0.0)
  z = jnp.dot(h, w2_ref[...], preferred_element_type=jnp.float32)
  y2_ref[...] = jnp.concatenate([dinv * z, jnp.zeros_like(z)], axis=1)


def _final_body(a_ref, y2_ref, dinv_ref, b2_ref, o_ref):
  o = dinv_ref[...] * (a_ref[...][:, :_O] + y2_ref[...][:, :_O]) + b2_ref[...]
  m = jnp.max(o, axis=1, keepdims=True)
  lse = jnp.log(jnp.sum(jnp.exp(o - m), axis=1, keepdims=True)) + m
  o_ref[...] = o - lse


def _rows(shape):
  return pl.BlockSpec(shape, lambda i: (i, 0))


def kernel(x, edge_index, W1, b1, W2, b2):
  src = edge_index[0].astype(jnp.int32)
  dst = edge_index[1].astype(jnp.int32)
  pad = _E_PAD - _E
  src2d = jnp.concatenate([src, jnp.zeros((pad,), jnp.int32)]).reshape(-1, _CHUNK)
  dst2d = jnp.concatenate([dst, jnp.full((pad,), _N, jnp.int32)]).reshape(-1, _CHUNK)

  z_h = jnp.zeros((_CHUNK, _H // 2), jnp.float32)
  z_o = jnp.zeros((_CHUNK, _O // 2), jnp.float32)
  z16 = jnp.zeros((_CHUNK, 16), jnp.float32)
  ones16 = jnp.ones((_CHUNK, 16), jnp.float32)

  grid = (_N // _RB,)

  degp = _sc_degree(dst2d, ones16, z16)

  xw1 = pl.pallas_call(
      _mm_body, grid=grid,
      in_specs=[_rows((_RB, _D)), pl.BlockSpec((_D, _H), lambda i: (0, 0))],
      out_specs=_rows((_RB, _H)),
      out_shape=jax.ShapeDtypeStruct((_N, _H), jnp.float32),
  )(x, W1)

  dinv, y1 = pl.pallas_call(
      _scale1_body, grid=grid,
      in_specs=[_rows((_RB, 128)), _rows((_RB, _H))],
      out_specs=(_rows((_RB, 1)), _rows((_RB, _H))),
      out_shape=(jax.ShapeDtypeStruct((_N, 1), jnp.float32),
                 jax.ShapeDtypeStruct((_N, _H), jnp.float32)),
  )(degp, xw1)

  acc1, _, _ = _sc_scatter_h(y1, src2d, dst2d, z_h)

  y2buf = pl.pallas_call(
      _layer2_body, grid=grid,
      in_specs=[_rows((_RB, _H)), _rows((_RB, _H)),
                _rows((_RB, 1)), pl.BlockSpec((1, _H), lambda i: (0, 0)),
                pl.BlockSpec((_H, _O), lambda i: (0, 0))],
      out_specs=_rows((_RB, 128)),
      out_shape=jax.ShapeDtypeStruct((_N, 128), jnp.float32),
  )(acc1, xw1, dinv, b1.reshape(1, _H), W2)

  acc2, _, _ = _sc_scatter_o(y2buf, src2d, dst2d, z_o)

  out = pl.pallas_call(
      _final_body, grid=grid,
      in_specs=[_rows((_RB, 128)), _rows((_RB, 128)),
                _rows((_RB, 1)), pl.BlockSpec((1, _O), lambda i: (0, 0))],
      out_specs=_rows((_RB, _O)),
      out_shape=jax.ShapeDtypeStruct((_N, _O), jnp.float32),
  )(acc2, y2buf, dinv, b2.reshape(1, _O))
  return out


# speedup vs baseline: 2.0664x; 1.0128x over previous
import functools

import jax
import jax.numpy as jnp
from jax import lax
from jax.experimental import pallas as pl
from jax.experimental.pallas import tpu as pltpu
from jax.experimental.pallas import tpu_sc as plsc

_N = 10000
_E = 320000
_D = 128
_H = 128
_O = 64

_NC = 2
_NT = 16
_NW = _NC * _NT

_CHUNK = 128
_N_PAD = 10240
_E_PAD = 327680
_CH_T = _E_PAD // (_NT * _CHUNK)
_SCH = 8
_NSCH = _CH_T // _SCH
_ZR = _N_PAD // _NT
_YR = _N // _NT
_WR = _YR // 5

_RB = 2000


def _make_sc_scatter(half, nbuf, kh0, kh1):
  mesh = plsc.VectorSubcoreMesh(core_axis_name="c", subcore_axis_name="s")

  @functools.partial(
      pl.kernel,
      out_type=(jax.ShapeDtypeStruct((_N, 128), jnp.float32),
                jax.ShapeDtypeStruct((_N, half), jnp.float32),
                jax.ShapeDtypeStruct((_N, half), jnp.float32)),
      mesh=mesh,
      compiler_params=pltpu.CompilerParams(use_tc_tiling_on_sc=False),
      scratch_types=[
          pltpu.VMEM((2, _SCH, _CHUNK), jnp.int32),
          pltpu.VMEM((2, _SCH, _CHUNK), jnp.int32),
          [pltpu.VMEM((_CHUNK, half), jnp.float32) for _ in range(nbuf)],
          pltpu.VMEM_SHARED((_N, half), jnp.float32),
          pltpu.VMEM_SHARED((_N_PAD, half), jnp.float32),
          [pltpu.SemaphoreType.DMA for _ in range(nbuf)],
          [pltpu.SemaphoreType.DMA for _ in range(nbuf)],
      ],
  )
  def scat(y_hbm, src_hbm, dst_hbm, z_hbm, out_hbm, yh0_hbm, yh1_hbm,
           src_v, dst_v, bufs, y_v, acc, gsems, ssems):
    c = lax.axis_index("c")
    s = lax.axis_index("s")
    yh = (yh0_hbm, yh1_hbm)
    with jax.named_scope("stage"):
      for cc in range(_NC):
        @pl.when(c == cc)
        def _(cc=cc):
          pltpu.sync_copy(
              y_hbm.at[pl.ds(s * _YR, _YR), pl.ds(cc * half, half)],
              y_v.at[pl.ds(s * _YR, _YR)])
          pltpu.sync_copy(y_v.at[pl.ds(s * _YR, _YR)],
                          yh[cc].at[pl.ds(s * _YR, _YR)])
      pltpu.sync_copy(z_hbm, bufs[0])
      for k in range(_ZR // _CHUNK):
        pltpu.sync_copy(bufs[0], acc.at[pl.ds(s * _ZR + k * _CHUNK, _CHUNK)])
      pltpu.sync_copy(src_hbm.at[pl.ds(s * _CH_T, _SCH)], src_v.at[0])
      pltpu.sync_copy(dst_hbm.at[pl.ds(s * _CH_T, _SCH)], dst_v.at[0])
      plsc.subcore_barrier()

    kh = jnp.where(c == 0, kh0, kh1)

    def g_fire(p, j, b, use_hbm):
      @pl.when(use_hbm)
      def _():
        for cc in range(_NC):
          @pl.when(c == cc)
          def _(cc=cc):
            pltpu.async_copy(yh[cc].at[src_v.at[p, j]], bufs[b], gsems[b])
      @pl.when(jnp.logical_not(use_hbm))
      def _():
        pltpu.async_copy(y_v.at[src_v.at[p, j]], bufs[b], gsems[b])

    def g_wait(p, j, b):
      pltpu.make_async_copy(y_v.at[src_v.at[p, j]], bufs[b], gsems[b]).wait()

    def s_fire(p, j, b):
      pltpu.async_copy(bufs[b], acc.at[dst_v.at[p, j]], ssems[b], add=True)

    def s_wait(p, j, b):
      pltpu.make_async_copy(bufs[b], acc.at[dst_v.at[p, j]], ssems[b]).wait()

    def outer(g, carry):
      p = lax.rem(g, 2)
      use_hbm = g < kh
      for b in range(nbuf):
        @pl.when(g > 0)
        def _(b=b):
          s_wait(1 - p, nbuf + b, b)
        g_fire(p, b, b, use_hbm)
      @pl.when(g + 1 < _NSCH)
      def _():
        base = s * _CH_T + (g + 1) * _SCH
        pltpu.sync_copy(src_hbm.at[pl.ds(base, _SCH)], src_v.at[1 - p])
        pltpu.sync_copy(dst_hbm.at[pl.ds(base, _SCH)], dst_v.at[1 - p])
      for b in range(nbuf):
        g_wait(p, b, b)
        s_fire(p, b, b)
      for b in range(nbuf):
        s_wait(p, b, b)
        g_fire(p, nbuf + b, b, use_hbm)
      for b in range(nbuf):
        g_wait(p, nbuf + b, b)
        s_fire(p, nbuf + b, b)
      return carry

    with jax.named_scope("edge_loop"):
      lax.fori_loop(0, _NSCH, outer, 0)
      p_last = lax.rem(_NSCH - 1, 2)
      for b in range(nbuf):
        s_wait(p_last, nbuf + b, b)
      plsc.subcore_barrier()
    with jax.named_scope("acc_writeout"):
      for k in range(_YR // _WR):
        r = s * _YR + k * _WR
        for cc in range(_NC):
          @pl.when(c == cc)
          def _(cc=cc, r=r):
            pltpu.sync_copy(acc.at[pl.ds(r, _WR)],
                            out_hbm.at[pl.ds(r, _WR), pl.ds(cc * half, half)])

  return scat


_sc_scatter_h = _make_sc_scatter(_H // 2, nbuf=4, kh0=9, kh1=3)
_sc_scatter_o = _make_sc_scatter(_O // 2, nbuf=4, kh0=9, kh1=3)


def _make_sc_degree():
  mesh = plsc.VectorSubcoreMesh(core_axis_name="c", subcore_axis_name="s")

  _CH_W = _E_PAD // (_NW * _CHUNK)

  @functools.partial(
      pl.kernel,
      out_type=jax.ShapeDtypeStruct((_N, 128), jnp.float32),
      mesh=mesh,
      compiler_params=pltpu.CompilerParams(use_tc_tiling_on_sc=False),
      scratch_types=[
          pltpu.VMEM((_E_PAD // (_NW * _CHUNK), _CHUNK), jnp.int32),
          pltpu.VMEM((_CHUNK, 16), jnp.float32),
          pltpu.VMEM((_CHUNK, 16), jnp.float32),
          pltpu.VMEM_SHARED((_N_PAD, 16), jnp.float32),
      ],
  )
  def degk(dst_hbm, ones_hbm, z_hbm, out_hbm, dst_v, ones_v, z_v, acc):
    c = lax.axis_index("c")
    s = lax.axis_index("s")
    row0 = (c * _NT + s) * _CH_W
    pltpu.sync_copy(dst_hbm.at[pl.ds(row0, _CH_W)], dst_v)
    pltpu.sync_copy(ones_hbm, ones_v)
    pltpu.sync_copy(z_hbm, z_v)
    for k in range(_ZR // _CHUNK):
      pltpu.sync_copy(z_v, acc.at[pl.ds(s * _ZR + k * _CHUNK, _CHUNK)])
    plsc.subcore_barrier()

    def step(j, carry):
      pltpu.sync_copy(ones_v, acc.at[dst_v.at[j]], add=True)
      return carry

    lax.fori_loop(0, _CH_W, step, 0)
    plsc.subcore_barrier()
    for k in range(_YR // _WR):
      r = s * _YR + k * _WR
      for cc in range(_NC):
        @pl.when(c == cc)
        def _(cc=cc, r=r):
          pltpu.sync_copy(acc.at[pl.ds(r, _WR)],
                          out_hbm.at[pl.ds(r, _WR), pl.ds(cc * 16, 16)])

  return degk


_sc_degree = _make_sc_degree()


def _mm_body(x_ref, w_ref, o_ref):
  o_ref[...] = jnp.dot(x_ref[...], w_ref[...],
                       preferred_element_type=jnp.float32)


def _scale1_body(dp_ref, xw_ref, dinv_ref, y_ref):
  dp = dp_ref[...]
  deg = dp[:, 0:1] + dp[:, 16:17] + 1.0
  dinv = lax.rsqrt(deg)
  dinv_ref[...] = dinv
  y_ref[...] = xw_ref[...] * dinv


def _layer2_body(a_ref, xw_ref, dinv_ref, b1_ref, w2_ref, y2_ref):
  dinv = dinv_ref[...]
  h = dinv * a_ref[...] + (dinv * dinv) * xw_ref[...]
  h = jnp.maximum(h + b1_ref[...], 0.0)
  z = jnp.dot(h, w2_ref[...], preferred_element_type=jnp.float32)
  y2_ref[...] = jnp.concatenate([dinv * z, jnp.zeros_like(z)], axis=1)


def _final_body(a_ref, y2_ref, dinv_ref, b2_ref, o_ref):
  o = dinv_ref[...] * (a_ref[...][:, :_O] + y2_ref[...][:, :_O]) + b2_ref[...]
  m = jnp.max(o, axis=1, keepdims=True)
  lse = jnp.log(jnp.sum(jnp.exp(o - m), axis=1, keepdims=True)) + m
  o_ref[...] = o - lse


def _rows(shape):
  return pl.BlockSpec(shape, lambda i: (i, 0))


def kernel(x, edge_index, W1, b1, W2, b2):
  src = edge_index[0].astype(jnp.int32)
  dst = edge_index[1].astype(jnp.int32)
  pad = _E_PAD - _E
  src2d = jnp.concatenate([src, jnp.zeros((pad,), jnp.int32)]).reshape(-1, _CHUNK)
  dst2d = jnp.concatenate([dst, jnp.full((pad,), _N, jnp.int32)]).reshape(-1, _CHUNK)

  z_h = jnp.zeros((_CHUNK, _H // 2), jnp.float32)
  z_o = jnp.zeros((_CHUNK, _O // 2), jnp.float32)
  z16 = jnp.zeros((_CHUNK, 16), jnp.float32)
  ones16 = jnp.ones((_CHUNK, 16), jnp.float32)

  grid = (_N // _RB,)

  degp = _sc_degree(dst2d, ones16, z16)

  xw1 = pl.pallas_call(
      _mm_body, grid=grid,
      in_specs=[_rows((_RB, _D)), pl.BlockSpec((_D, _H), lambda i: (0, 0))],
      out_specs=_rows((_RB, _H)),
      out_shape=jax.ShapeDtypeStruct((_N, _H), jnp.float32),
  )(x, W1)

  dinv, y1 = pl.pallas_call(
      _scale1_body, grid=grid,
      in_specs=[_rows((_RB, 128)), _rows((_RB, _H))],
      out_specs=(_rows((_RB, 1)), _rows((_RB, _H))),
      out_shape=(jax.ShapeDtypeStruct((_N, 1), jnp.float32),
                 jax.ShapeDtypeStruct((_N, _H), jnp.float32)),
  )(degp, xw1)

  acc1, _, _ = _sc_scatter_h(y1, src2d, dst2d, z_h)

  y2buf = pl.pallas_call(
      _layer2_body, grid=grid,
      in_specs=[_rows((_RB, _H)), _rows((_RB, _H)),
                _rows((_RB, 1)), pl.BlockSpec((1, _H), lambda i: (0, 0)),
                pl.BlockSpec((_H, _O), lambda i: (0, 0))],
      out_specs=_rows((_RB, 128)),
      out_shape=jax.ShapeDtypeStruct((_N, 128), jnp.float32),
  )(acc1, xw1, dinv, b1.reshape(1, _H), W2)

  acc2, _, _ = _sc_scatter_o(y2buf, src2d, dst2d, z_o)

  out = pl.pallas_call(
      _final_body, grid=grid,
      in_specs=[_rows((_RB, 128)), _rows((_RB, 128)),
                _rows((_RB, 1)), pl.BlockSpec((1, _O), lambda i: (0, 0))],
      out_specs=_rows((_RB, _O)),
      out_shape=jax.ShapeDtypeStruct((_N, _O), jnp.float32),
  )(acc2, y2buf, dinv, b2.reshape(1, _O))
  return out


# speedup vs baseline: 2.0849x; 1.0090x over previous
import functools

import jax
import jax.numpy as jnp
from jax import lax
from jax.experimental import pallas as pl
from jax.experimental.pallas import tpu as pltpu
from jax.experimental.pallas import tpu_sc as plsc

_N = 10000
_E = 320000
_D = 128
_H = 128
_O = 64

_NC = 2
_NT = 16
_NW = _NC * _NT

_CHUNK = 128
_N_PAD = 10240
_E_PAD = 327680
_CH_T = _E_PAD // (_NT * _CHUNK)
_SCH = 8
_NSCH = _CH_T // _SCH
_ZR = _N_PAD // _NT
_YR = _N // _NT
_WR = _YR // 5

_RB = 2000


def _make_sc_scatter(half, nbuf, kh0, kh1):
  mesh = plsc.VectorSubcoreMesh(core_axis_name="c", subcore_axis_name="s")

  @functools.partial(
      pl.kernel,
      out_type=(jax.ShapeDtypeStruct((_N, 128), jnp.float32),
                jax.ShapeDtypeStruct((_N, half), jnp.float32),
                jax.ShapeDtypeStruct((_N, half), jnp.float32)),
      mesh=mesh,
      compiler_params=pltpu.CompilerParams(use_tc_tiling_on_sc=False),
      scratch_types=[
          pltpu.VMEM((2, _SCH, _CHUNK), jnp.int32),
          pltpu.VMEM((2, _SCH, _CHUNK), jnp.int32),
          [pltpu.VMEM((_CHUNK, half), jnp.float32) for _ in range(nbuf)],
          pltpu.VMEM_SHARED((_N, half), jnp.float32),
          pltpu.VMEM_SHARED((_N_PAD, half), jnp.float32),
          [pltpu.SemaphoreType.DMA for _ in range(nbuf)],
          [pltpu.SemaphoreType.DMA for _ in range(nbuf)],
      ],
  )
  def scat(y_hbm, src_hbm, dst_hbm, z_hbm, out_hbm, yh0_hbm, yh1_hbm,
           src_v, dst_v, bufs, y_v, acc, gsems, ssems):
    c = lax.axis_index("c")
    s = lax.axis_index("s")
    yh = (yh0_hbm, yh1_hbm)
    with jax.named_scope("stage"):
      for cc in range(_NC):
        @pl.when(c == cc)
        def _(cc=cc):
          pltpu.sync_copy(
              y_hbm.at[pl.ds(s * _YR, _YR), pl.ds(cc * half, half)],
              y_v.at[pl.ds(s * _YR, _YR)])
          pltpu.sync_copy(y_v.at[pl.ds(s * _YR, _YR)],
                          yh[cc].at[pl.ds(s * _YR, _YR)])
      pltpu.sync_copy(z_hbm, bufs[0])
      for k in range(_ZR // _CHUNK):
        pltpu.sync_copy(bufs[0], acc.at[pl.ds(s * _ZR + k * _CHUNK, _CHUNK)])
      pltpu.sync_copy(src_hbm.at[pl.ds(s * _CH_T, _SCH)], src_v.at[0])
      pltpu.sync_copy(dst_hbm.at[pl.ds(s * _CH_T, _SCH)], dst_v.at[0])
      plsc.subcore_barrier()

    kh = jnp.where(c == 0, kh0, kh1)

    def g_fire(p, j, b, use_hbm):
      @pl.when(use_hbm)
      def _():
        for cc in range(_NC):
          @pl.when(c == cc)
          def _(cc=cc):
            pltpu.async_copy(yh[cc].at[src_v.at[p, j]], bufs[b], gsems[b])
      @pl.when(jnp.logical_not(use_hbm))
      def _():
        pltpu.async_copy(y_v.at[src_v.at[p, j]], bufs[b], gsems[b])

    def g_wait(p, j, b):
      pltpu.make_async_copy(y_v.at[src_v.at[p, j]], bufs[b], gsems[b]).wait()

    def s_fire(p, j, b):
      pltpu.async_copy(bufs[b], acc.at[dst_v.at[p, j]], ssems[b], add=True)

    def s_wait(p, j, b):
      pltpu.make_async_copy(bufs[b], acc.at[dst_v.at[p, j]], ssems[b]).wait()

    def outer(g, carry):
      p = lax.rem(g, 2)
      use_hbm = g < kh
      for b in range(nbuf):
        @pl.when(g > 0)
        def _(b=b):
          s_wait(1 - p, nbuf + b, b)
        g_fire(p, b, b, use_hbm)
      @pl.when(g + 1 < _NSCH)
      def _():
        base = s * _CH_T + (g + 1) * _SCH
        pltpu.sync_copy(src_hbm.at[pl.ds(base, _SCH)], src_v.at[1 - p])
        pltpu.sync_copy(dst_hbm.at[pl.ds(base, _SCH)], dst_v.at[1 - p])
      for b in range(nbuf):
        g_wait(p, b, b)
        s_fire(p, b, b)
      for b in range(nbuf):
        s_wait(p, b, b)
        g_fire(p, nbuf + b, b, use_hbm)
      for b in range(nbuf):
        g_wait(p, nbuf + b, b)
        s_fire(p, nbuf + b, b)
      return carry

    with jax.named_scope("edge_loop"):
      lax.fori_loop(0, _NSCH, outer, 0)
      p_last = lax.rem(_NSCH - 1, 2)
      for b in range(nbuf):
        s_wait(p_last, nbuf + b, b)
      plsc.subcore_barrier()
    with jax.named_scope("acc_writeout"):
      for k in range(_YR // _WR):
        r = s * _YR + k * _WR
        for cc in range(_NC):
          @pl.when(c == cc)
          def _(cc=cc, r=r):
            pltpu.sync_copy(acc.at[pl.ds(r, _WR)],
                            out_hbm.at[pl.ds(r, _WR), pl.ds(cc * half, half)])

  return scat


_sc_scatter_h = _make_sc_scatter(_H // 2, nbuf=4, kh0=12, kh1=4)
_sc_scatter_o = _make_sc_scatter(_O // 2, nbuf=4, kh0=12, kh1=4)


def _make_sc_degree():
  mesh = plsc.VectorSubcoreMesh(core_axis_name="c", subcore_axis_name="s")

  _CH_W = _E_PAD // (_NW * _CHUNK)

  @functools.partial(
      pl.kernel,
      out_type=jax.ShapeDtypeStruct((_N, 128), jnp.float32),
      mesh=mesh,
      compiler_params=pltpu.CompilerParams(use_tc_tiling_on_sc=False),
      scratch_types=[
          pltpu.VMEM((_E_PAD // (_NW * _CHUNK), _CHUNK), jnp.int32),
          pltpu.VMEM((_CHUNK, 16), jnp.float32),
          pltpu.VMEM((_CHUNK, 16), jnp.float32),
          pltpu.VMEM_SHARED((_N_PAD, 16), jnp.float32),
      ],
  )
  def degk(dst_hbm, ones_hbm, z_hbm, out_hbm, dst_v, ones_v, z_v, acc):
    c = lax.axis_index("c")
    s = lax.axis_index("s")
    row0 = (c * _NT + s) * _CH_W
    pltpu.sync_copy(dst_hbm.at[pl.ds(row0, _CH_W)], dst_v)
    pltpu.sync_copy(ones_hbm, ones_v)
    pltpu.sync_copy(z_hbm, z_v)
    for k in range(_ZR // _CHUNK):
      pltpu.sync_copy(z_v, acc.at[pl.ds(s * _ZR + k * _CHUNK, _CHUNK)])
    plsc.subcore_barrier()

    def step(j, carry):
      pltpu.sync_copy(ones_v, acc.at[dst_v.at[j]], add=True)
      return carry

    lax.fori_loop(0, _CH_W, step, 0)
    plsc.subcore_barrier()
    for k in range(_YR // _WR):
      r = s * _YR + k * _WR
      for cc in range(_NC):
        @pl.when(c == cc)
        def _(cc=cc, r=r):
          pltpu.sync_copy(acc.at[pl.ds(r, _WR)],
                          out_hbm.at[pl.ds(r, _WR), pl.ds(cc * 16, 16)])

  return degk


_sc_degree = _make_sc_degree()


def _mm_body(x_ref, w_ref, o_ref):
  o_ref[...] = jnp.dot(x_ref[...], w_ref[...],
                       preferred_element_type=jnp.float32)


def _scale1_body(dp_ref, xw_ref, dinv_ref, y_ref):
  dp = dp_ref[...]
  deg = dp[:, 0:1] + dp[:, 16:17] + 1.0
  dinv = lax.rsqrt(deg)
  dinv_ref[...] = dinv
  y_ref[...] = xw_ref[...] * dinv


def _layer2_body(a_ref, xw_ref, dinv_ref, b1_ref, w2_ref, y2_ref):
  dinv = dinv_ref[...]
  h = dinv * a_ref[...] + (dinv * dinv) * xw_ref[...]
  h = jnp.maximum(h + b1_ref[...], 0.0)
  z = jnp.dot(h, w2_ref[...], preferred_element_type=jnp.float32)
  y2_ref[...] = jnp.concatenate([dinv * z, jnp.zeros_like(z)], axis=1)


def _final_body(a_ref, y2_ref, dinv_ref, b2_ref, o_ref):
  o = dinv_ref[...] * (a_ref[...][:, :_O] + y2_ref[...][:, :_O]) + b2_ref[...]
  m = jnp.max(o, axis=1, keepdims=True)
  lse = jnp.log(jnp.sum(jnp.exp(o - m), axis=1, keepdims=True)) + m
  o_ref[...] = o - lse


def _rows(shape):
  return pl.BlockSpec(shape, lambda i: (i, 0))


def kernel(x, edge_index, W1, b1, W2, b2):
  src = edge_index[0].astype(jnp.int32)
  dst = edge_index[1].astype(jnp.int32)
  pad = _E_PAD - _E
  src2d = jnp.concatenate([src, jnp.zeros((pad,), jnp.int32)]).reshape(-1, _CHUNK)
  dst2d = jnp.concatenate([dst, jnp.full((pad,), _N, jnp.int32)]).reshape(-1, _CHUNK)

  z_h = jnp.zeros((_CHUNK, _H // 2), jnp.float32)
  z_o = jnp.zeros((_CHUNK, _O // 2), jnp.float32)
  z16 = jnp.zeros((_CHUNK, 16), jnp.float32)
  ones16 = jnp.ones((_CHUNK, 16), jnp.float32)

  grid = (_N // _RB,)

  degp = _sc_degree(dst2d, ones16, z16)

  xw1 = pl.pallas_call(
      _mm_body, grid=grid,
      in_specs=[_rows((_RB, _D)), pl.BlockSpec((_D, _H), lambda i: (0, 0))],
      out_specs=_rows((_RB, _H)),
      out_shape=jax.ShapeDtypeStruct((_N, _H), jnp.float32),
  )(x, W1)

  dinv, y1 = pl.pallas_call(
      _scale1_body, grid=grid,
      in_specs=[_rows((_RB, 128)), _rows((_RB, _H))],
      out_specs=(_rows((_RB, 1)), _rows((_RB, _H))),
      out_shape=(jax.ShapeDtypeStruct((_N, 1), jnp.float32),
                 jax.ShapeDtypeStruct((_N, _H), jnp.float32)),
  )(degp, xw1)

  acc1, _, _ = _sc_scatter_h(y1, src2d, dst2d, z_h)

  y2buf = pl.pallas_call(
      _layer2_body, grid=grid,
      in_specs=[_rows((_RB, _H)), _rows((_RB, _H)),
                _rows((_RB, 1)), pl.BlockSpec((1, _H), lambda i: (0, 0)),
                pl.BlockSpec((_H, _O), lambda i: (0, 0))],
      out_specs=_rows((_RB, 128)),
      out_shape=jax.ShapeDtypeStruct((_N, 128), jnp.float32),
  )(acc1, xw1, dinv, b1.reshape(1, _H), W2)

  acc2, _, _ = _sc_scatter_o(y2buf, src2d, dst2d, z_o)

  out = pl.pallas_call(
      _final_body, grid=grid,
      in_specs=[_rows((_RB, 128)), _rows((_RB, 128)),
                _rows((_RB, 1)), pl.BlockSpec((1, _O), lambda i: (0, 0))],
      out_specs=_rows((_RB, _O)),
      out_shape=jax.ShapeDtypeStruct((_N, _O), jnp.float32),
  )(acc2, y2buf, dinv, b2.reshape(1, _O))
  return out
